# baseline (speedup 1.0000x reference)
import functools

import jax
import jax.numpy as jnp
from jax import lax
from jax.experimental import pallas as pl
from jax.experimental.pallas import tpu as pltpu
from jax.experimental.pallas import tpu_sc as plsc

_C = 2048
_CS = 128
_NW = 32


@functools.lru_cache(maxsize=None)
def _build_interp(n_points, vol_shape):
    D, H, W = vol_shape
    n_per_w = n_points // _NW
    T = n_per_w // _C
    assert T % 2 == 0
    mesh = plsc.VectorSubcoreMesh(core_axis_name="c", subcore_axis_name="s")

    @functools.partial(
        pl.kernel,
        out_type=jax.ShapeDtypeStruct((n_points,), jnp.float32),
        mesh=mesh,
        compiler_params=pltpu.CompilerParams(needs_layout_passes=False),
        scratch_types=[
            pltpu.VMEM((_CS, 3), jnp.float32),
            pltpu.VMEM((_CS, 3), jnp.float32),
            pltpu.VMEM((_C,), jnp.float32),
            pltpu.VMEM((_C,), jnp.float32),
            pltpu.VMEM((_C,), jnp.float32),
            pltpu.VMEM((8 * _C,), jnp.int32),
            pltpu.VMEM((8 * _C,), jnp.int32),
            pltpu.VMEM((8 * _C,), jnp.float32),
            pltpu.VMEM((8 * _C,), jnp.float32),
            pltpu.VMEM((3 * _C,), jnp.float32),
            pltpu.VMEM((3 * _C,), jnp.float32),
            pltpu.VMEM((_C,), jnp.float32),
            pltpu.SemaphoreType.DMA,
            pltpu.SemaphoreType.DMA,
            pltpu.SemaphoreType.DMA,
            pltpu.SemaphoreType.DMA,
        ],
    )
    def interp_kernel(coords_hbm, lin_hbm, out_hbm,
                      cv_a, cv_b, z_v, y_v, x_v, idx_a, idx_b, g_a, g_b,
                      w_a, w_b, o_v, sem_a, sem_b, cs_a, cs_b):
        wid = lax.axis_index("s") * 2 + lax.axis_index("c")
        wbase = wid * n_per_w
        lane = lax.iota(jnp.int32, 16)
        zero16 = lane * 0

        cbufs = None

        def load_coords(t):
            base = wbase + t * _C
            nsub = _C // _CS
            pltpu.async_copy(coords_hbm.at[pl.ds(base, _CS), :], cv_a, cs_a)

            def csub2(s2, carry0):
                for cpar in (0, 1):
                    s = s2 * 2 + cpar
                    cv_cur, cs_cur = cbufs[cpar]
                    cv_nxt, cs_nxt = cbufs[1 - cpar]

                    @pl.when(s + 1 < nsub)
                    def _():
                        pltpu.async_copy(
                            coords_hbm.at[pl.ds(base + (s + 1) * _CS, _CS), :],
                            cv_nxt, cs_nxt,
                        )

                    pltpu.make_async_copy(
                        coords_hbm.at[pl.ds(base + s * _CS, _CS), :],
                        cv_cur, cs_cur,
                    ).wait()

                    def grp0(g, carry00):
                        p = g * 16
                        q = s * _CS + p
                        z_v[pl.ds(q, 16)] = plsc.load_gather(cv_cur, [p + lane, zero16])
                        y_v[pl.ds(q, 16)] = plsc.load_gather(cv_cur, [p + lane, zero16 + 1])
                        x_v[pl.ds(q, 16)] = plsc.load_gather(cv_cur, [p + lane, zero16 + 2])
                        return carry00

                    lax.fori_loop(0, _CS // 16, grp0, 0)
                return carry0

            lax.fori_loop(0, _C // (2 * _CS), csub2, 0)

        def p1(idx_v, w_v):
            def body(g, carry1):
                p = g * 16
                sz = z_v[pl.ds(p, 16)] * float(D - 1)
                sy = y_v[pl.ds(p, 16)] * float(H - 1)
                sx = x_v[pl.ds(p, 16)] * float(W - 1)
                iz = jnp.minimum(sz.astype(jnp.int32), D - 2)
                iy = jnp.minimum(sy.astype(jnp.int32), H - 2)
                ix = jnp.minimum(sx.astype(jnp.int32), W - 2)
                w_v[pl.ds(p, 16)] = sz - iz.astype(jnp.float32)
                w_v[pl.ds(_C + p, 16)] = sy - iy.astype(jnp.float32)
                w_v[pl.ds(2 * _C + p, 16)] = sx - ix.astype(jnp.float32)
                l = (iz * H + iy) * W + ix
                e = (p + lane) * 8
                plsc.store_scatter(idx_v, [e], l)
                plsc.store_scatter(idx_v, [e + 1], l + 1)
                plsc.store_scatter(idx_v, [e + 2], l + W)
                plsc.store_scatter(idx_v, [e + 3], l + (W + 1))
                plsc.store_scatter(idx_v, [e + 4], l + H * W)
                plsc.store_scatter(idx_v, [e + 5], l + (H * W + 1))
                plsc.store_scatter(idx_v, [e + 6], l + (H * W + W))
                plsc.store_scatter(idx_v, [e + 7], l + (H * W + W + 1))
                return carry1

            lax.fori_loop(0, _C // 16, body, 0)

        def p2(t, g_v, w_v):
            base = wbase + t * _C

            def body(g, carry2):
                p = g * 16
                wz = w_v[pl.ds(p, 16)]
                wy = w_v[pl.ds(_C + p, 16)]
                wx = w_v[pl.ds(2 * _C + p, 16)]
                e = (p + lane) * 8
                c000 = plsc.load_gather(g_v, [e])
                c001 = plsc.load_gather(g_v, [e + 1])
                c010 = plsc.load_gather(g_v, [e + 2])
                c011 = plsc.load_gather(g_v, [e + 3])
                c100 = plsc.load_gather(g_v, [e + 4])
                c101 = plsc.load_gather(g_v, [e + 5])
                c110 = plsc.load_gather(g_v, [e + 6])
                c111 = plsc.load_gather(g_v, [e + 7])
                c00 = c000 + wx * (c001 - c000)
                c01 = c010 + wx * (c011 - c010)
                c10 = c100 + wx * (c101 - c100)
                c11 = c110 + wx * (c111 - c110)
                c0 = c00 + wy * (c01 - c00)
                c1 = c10 + wy * (c11 - c10)
                o_v[pl.ds(p, 16)] = c0 + wz * (c1 - c0)
                return carry2

            lax.fori_loop(0, _C // 16, body, 0)
            pltpu.sync_copy(o_v, out_hbm.at[pl.ds(base, _C)])

        bufs = (
            (idx_a, g_a, w_a, sem_a),
            (idx_b, g_b, w_b, sem_b),
        )
        cbufs = ((cv_a, cs_a), (cv_b, cs_b))

        load_coords(0)
        p1(idx_a, w_a)
        pltpu.async_copy(lin_hbm.at[idx_a], g_a, sem_a)

        def outer(t2, carry):
            for par in (0, 1):
                t = t2 * 2 + par
                idx_cur, g_cur, w_cur, sem_cur = bufs[par]
                idx_nxt, g_nxt, w_nxt, sem_nxt = bufs[1 - par]

                @pl.when(t + 1 < T)
                def _():
                    load_coords(t + 1)
                    p1(idx_nxt, w_nxt)
                    pltpu.async_copy(lin_hbm.at[idx_nxt], g_nxt, sem_nxt)

                pltpu.make_async_copy(lin_hbm.at[idx_cur], g_cur, sem_cur).wait()
                p2(t, g_cur, w_cur)
            return carry

        lax.fori_loop(0, T // 2, outer, 0)

    return interp_kernel


def kernel(coords, data):
    n = coords.shape[0]
    lin = jnp.maximum(data.reshape(-1), jnp.float32(0.0))
    out = _build_interp(n, data.shape)(coords, lin)
    return out.reshape(n, 1)


# speedup vs baseline: 1.1892x; 1.1892x over previous
import functools

import jax
import jax.numpy as jnp
from jax import lax
from jax.experimental import pallas as pl
from jax.experimental.pallas import tpu as pltpu
from jax.experimental.pallas import tpu_sc as plsc

_C = 2048
_CS = 128
_NW = 32


@functools.lru_cache(maxsize=None)
def _build_interp(n_points, vol_shape):
    D, H, W = vol_shape
    n_per_w = n_points // _NW
    T = n_per_w // _C
    assert T % 2 == 0
    mesh = plsc.VectorSubcoreMesh(core_axis_name="c", subcore_axis_name="s")

    @functools.partial(
        pl.kernel,
        out_type=jax.ShapeDtypeStruct((n_points,), jnp.float32),
        mesh=mesh,
        compiler_params=pltpu.CompilerParams(needs_layout_passes=False),
        scratch_types=[
            pltpu.VMEM((_CS, 3), jnp.float32),
            pltpu.VMEM((_CS, 3), jnp.float32),
            pltpu.VMEM((_C,), jnp.float32),
            pltpu.VMEM((_C,), jnp.float32),
            pltpu.VMEM((_C,), jnp.float32),
            pltpu.VMEM((8 * _C,), jnp.int32),
            pltpu.VMEM((8 * _C,), jnp.int32),
            pltpu.VMEM((8 * _C,), jnp.float32),
            pltpu.VMEM((8 * _C,), jnp.float32),
            pltpu.VMEM((3 * _C,), jnp.float32),
            pltpu.VMEM((3 * _C,), jnp.float32),
            pltpu.VMEM((_C,), jnp.float32),
            pltpu.SemaphoreType.DMA,
            pltpu.SemaphoreType.DMA,
            pltpu.SemaphoreType.DMA,
            pltpu.SemaphoreType.DMA,
        ],
    )
    def interp_kernel(coords_hbm, lin_hbm, out_hbm,
                      cv_a, cv_b, z_v, y_v, x_v, idx_a, idx_b, g_a, g_b,
                      w_a, w_b, o_v, sem_a, sem_b, cs_a, cs_b):
        wid = lax.axis_index("s") * 2 + lax.axis_index("c")
        wbase = wid * n_per_w
        lane = lax.iota(jnp.int32, 16)
        zero16 = lane * 0

        cbufs = None

        def load_coords(t):
            base = wbase + t * _C
            nsub = _C // _CS
            pltpu.async_copy(coords_hbm.at[pl.ds(base, _CS), :], cv_a, cs_a)

            def csub2(s2, carry0):
                for cpar in (0, 1):
                    s = s2 * 2 + cpar
                    cv_cur, cs_cur = cbufs[cpar]
                    cv_nxt, cs_nxt = cbufs[1 - cpar]

                    @pl.when(s + 1 < nsub)
                    def _():
                        pltpu.async_copy(
                            coords_hbm.at[pl.ds(base + (s + 1) * _CS, _CS), :],
                            cv_nxt, cs_nxt,
                        )

                    pltpu.make_async_copy(
                        coords_hbm.at[pl.ds(base + s * _CS, _CS), :],
                        cv_cur, cs_cur,
                    ).wait()

                    def grp0(g, carry00):
                        p = g * 16
                        q = s * _CS + p
                        z_v[pl.ds(q, 16)] = plsc.load_gather(cv_cur, [p + lane, zero16])
                        y_v[pl.ds(q, 16)] = plsc.load_gather(cv_cur, [p + lane, zero16 + 1])
                        x_v[pl.ds(q, 16)] = plsc.load_gather(cv_cur, [p + lane, zero16 + 2])
                        return carry00

                    lax.fori_loop(0, _CS // 16, grp0, 0)
                return carry0

            lax.fori_loop(0, _C // (2 * _CS), csub2, 0)

        def p1(idx_v, w_v):
            def body(g, carry1):
                p = g * 16
                sz = z_v[pl.ds(p, 16)] * float(D - 1)
                sy = y_v[pl.ds(p, 16)] * float(H - 1)
                sx = x_v[pl.ds(p, 16)] * float(W - 1)
                iz = jnp.minimum(sz.astype(jnp.int32), D - 2)
                iy = jnp.minimum(sy.astype(jnp.int32), H - 2)
                ix = jnp.minimum(sx.astype(jnp.int32), W - 2)
                w_v[pl.ds(p, 16)] = sz - iz.astype(jnp.float32)
                w_v[pl.ds(_C + p, 16)] = sy - iy.astype(jnp.float32)
                w_v[pl.ds(2 * _C + p, 16)] = sx - ix.astype(jnp.float32)
                l = (iz * H + iy) * W + ix
                e = (p + lane) * 8
                plsc.store_scatter(idx_v, [e], l)
                plsc.store_scatter(idx_v, [e + 1], l + 1)
                plsc.store_scatter(idx_v, [e + 2], l + W)
                plsc.store_scatter(idx_v, [e + 3], l + (W + 1))
                plsc.store_scatter(idx_v, [e + 4], l + H * W)
                plsc.store_scatter(idx_v, [e + 5], l + (H * W + 1))
                plsc.store_scatter(idx_v, [e + 6], l + (H * W + W))
                plsc.store_scatter(idx_v, [e + 7], l + (H * W + W + 1))
                return carry1

            lax.fori_loop(0, _C // 16, body, 0)

        def p2(t, g_v, w_v):
            base = wbase + t * _C

            def body(g, carry2):
                p = g * 16
                wz = w_v[pl.ds(p, 16)]
                wy = w_v[pl.ds(_C + p, 16)]
                wx = w_v[pl.ds(2 * _C + p, 16)]
                e = (p + lane) * 8
                c000 = plsc.load_gather(g_v, [e])
                c001 = plsc.load_gather(g_v, [e + 1])
                c010 = plsc.load_gather(g_v, [e + 2])
                c011 = plsc.load_gather(g_v, [e + 3])
                c100 = plsc.load_gather(g_v, [e + 4])
                c101 = plsc.load_gather(g_v, [e + 5])
                c110 = plsc.load_gather(g_v, [e + 6])
                c111 = plsc.load_gather(g_v, [e + 7])
                c00 = c000 + wx * (c001 - c000)
                c01 = c010 + wx * (c011 - c010)
                c10 = c100 + wx * (c101 - c100)
                c11 = c110 + wx * (c111 - c110)
                c0 = c00 + wy * (c01 - c00)
                c1 = c10 + wy * (c11 - c10)
                o_v[pl.ds(p, 16)] = c0 + wz * (c1 - c0)
                return carry2

            lax.fori_loop(0, _C // 16, body, 0)
            pltpu.sync_copy(o_v, out_hbm.at[pl.ds(base, _C)])

        bufs = (
            (idx_a, g_a, w_a, sem_a),
            (idx_b, g_b, w_b, sem_b),
        )
        cbufs = ((cv_a, cs_a), (cv_b, cs_b))

        load_coords(0)
        p1(idx_a, w_a)
        pltpu.async_copy(lin_hbm.at[idx_a], g_a, sem_a)

        def outer(t2, carry):
            for par in (0, 1):
                t = t2 * 2 + par
                idx_cur, g_cur, w_cur, sem_cur = bufs[par]
                idx_nxt, g_nxt, w_nxt, sem_nxt = bufs[1 - par]

                @pl.when(t + 1 < T)
                def _():
                    load_coords(t + 1)
                    p1(idx_nxt, w_nxt)
                    pltpu.async_copy(lin_hbm.at[idx_nxt], g_nxt, sem_nxt)

                pltpu.make_async_copy(lin_hbm.at[idx_cur], g_cur, sem_cur).wait()
                p2(t, g_cur, w_cur)
            return carry

        lax.fori_loop(0, T // 2, outer, 0)

    return interp_kernel


def kernel(coords, data):
    n = coords.shape[0]
    flat = data.reshape(-1)
    guard = lax.broadcasted_iota(jnp.int32, flat.shape, 0) >= 0
    lin = jnp.where(guard, flat, jnp.float32(0.0))
    out = _build_interp(n, data.shape)(coords, lin)
    return out.reshape(n, 1)
